# baseline (speedup 1.0000x reference)
import functools

import jax
import jax.numpy as jnp
from jax import lax
from jax.experimental import pallas as pl
from jax.experimental.pallas import tpu as pltpu
from jax.experimental.pallas import tpu_sc as plsc

N = 10000
E = 320000
NPAD = 10240
CHUNK = 128
NC = 2
NS = 16
NW = NC * NS
NBUF = 2
NCHUNKS = 80
G = NCHUNKS // NBUF
EPW = NCHUNKS * CHUNK
EP = NW * EPW
RPT = NPAD // NS
EPS = 1e-5
RBLK = 2048
GRID = NPAD // RBLK

def _zero_fill(ref, d):
    z = jnp.zeros((16,), jnp.float32)
    for r in range(16):
        for c in range(d // 16):
            ref[r, pl.ds(c * 16, 16)] = z


def _zero_acc(acc, zbuf, sid, d):
    _zero_fill(zbuf, d)

    def zloop(i, _):
        pltpu.sync_copy(zbuf, acc.at[pl.ds(sid * RPT + i * 16, 16)])
        return 0

    lax.fori_loop(0, RPT // 16, zloop, 0)


@functools.cache
def _sc_kernels():
    mesh = plsc.VectorSubcoreMesh(core_axis_name="c", subcore_axis_name="s",
                                  num_cores=NC, num_subcores=NS)

    @functools.partial(
        pl.kernel,
        out_type=jax.ShapeDtypeStruct((NC * NPAD, 128), jnp.float32),
        mesh=mesh,
        scratch_types=[
            pltpu.VMEM_SHARED((NPAD, 128), jnp.float32),
            pltpu.VMEM((NCHUNKS, CHUNK), jnp.int32),
            pltpu.VMEM((CHUNK, 128), jnp.float32),
            pltpu.VMEM((16, 128), jnp.float32),
        ],
    )
    def _sc_hist(dst_hbm, out_hbm, acc, dbuf, ones, zbuf):
        cid = lax.axis_index("c")
        sid = lax.axis_index("s")
        _zero_acc(acc, zbuf, sid, 128)
        one = jnp.ones((16,), jnp.float32)
        for r in range(CHUNK):
            for c in range(8):
                ones[r, pl.ds(c * 16, 16)] = one
        wid = cid * NS + sid
        pltpu.sync_copy(dst_hbm.at[pl.ds(wid * NCHUNKS, NCHUNKS)], dbuf)
        plsc.subcore_barrier()

        def body(i, _):
            pltpu.sync_copy(ones, acc.at[dbuf.at[i]], add=True)
            return 0

        lax.fori_loop(0, NCHUNKS, body, 0)
        plsc.subcore_barrier()
        row0 = sid * RPT
        pltpu.sync_copy(acc.at[pl.ds(row0, RPT)],
                        out_hbm.at[pl.ds(cid * NPAD + row0, RPT)])

    def _make_sc_scatter(d):
        @functools.partial(
            pl.kernel,
            out_type=jax.ShapeDtypeStruct((NC * NPAD, d), jnp.float32),
            mesh=mesh,
            scratch_types=[
                pltpu.VMEM_SHARED((NPAD, d), jnp.float32),
                pltpu.VMEM((2 * NCHUNKS, CHUNK), jnp.int32),
                pltpu.VMEM((CHUNK, d), jnp.float32),
                pltpu.VMEM((16, d), jnp.float32),
                pltpu.SemaphoreType.DMA,
            ],
        )
        def _sc_scatter(idx_hbm, tbl_hbm, out_hbm,
                        acc, ibuf, rows, zbuf, gsem):
            cid = lax.axis_index("c")
            sid = lax.axis_index("s")
            _zero_acc(acc, zbuf, sid, d)
            wid = cid * NS + sid
            pltpu.sync_copy(idx_hbm.at[pl.ds(wid * 2 * NCHUNKS, 2 * NCHUNKS)],
                            ibuf)
            plsc.subcore_barrier()

            def body(i, _):
                pltpu.async_copy(tbl_hbm.at[ibuf.at[2 * i]], rows, gsem).wait()
                pltpu.sync_copy(rows, acc.at[ibuf.at[2 * i + 1]], add=True)
                return 0

            lax.fori_loop(0, NCHUNKS, body, 0)
            plsc.subcore_barrier()
            row0 = sid * RPT
            pltpu.sync_copy(acc.at[pl.ds(row0, RPT)],
                            out_hbm.at[pl.ds(cid * NPAD + row0, RPT)])

        return _sc_scatter

    return _sc_hist, _make_sc_scatter(128)


def _first_kernel(x_ref, w_ref, cnt0_ref, cnt1_ref, out_ref, dis_ref):
    c = cnt0_ref[...][:, 0:1] + cnt1_ref[...][:, 0:1]
    dis = lax.rsqrt(1.0 + c)
    dis_ref[...] = jnp.broadcast_to(dis, (RBLK, 8))
    out_ref[...] = dis * jnp.dot(x_ref[...], w_ref[...],
                                 preferred_element_type=jnp.float32,
                                 precision=lax.Precision.HIGHEST)


def _mid_kernel(dv, p0_ref, p1_ref, hp_ref, dis_ref,
                w_ref, b_ref, g_ref, be_ref, out_ref):
    dis = dis_ref[...][:, 0:1]
    v = (p0_ref[...] + p1_ref[...] + hp_ref[...])[:, :dv]
    conv = dis * v + b_ref[...]
    bnscale = 1.0 / jnp.sqrt(jnp.float32(1.0 + EPS))
    y = jnp.maximum(conv * bnscale * g_ref[...] + be_ref[...], 0.0)
    r = dis * jnp.dot(y, w_ref[...],
                      preferred_element_type=jnp.float32,
                      precision=lax.Precision.HIGHEST)
    out_ref[...] = jnp.concatenate(
        [r, jnp.zeros((RBLK, 128 - r.shape[1]), jnp.float32)], axis=1)


def _last_kernel(p0_ref, p1_ref, hp_ref, dis_ref, b_ref, out_ref):
    dis = dis_ref[...][:, 0:1]
    v = (p0_ref[...] + p1_ref[...] + hp_ref[...])[:, :64]
    out_ref[...] = dis * v + b_ref[...]


def _row_spec(d):
    return pl.BlockSpec((RBLK, d), lambda i: (i, 0))


def _p0_spec(d):
    return pl.BlockSpec((RBLK, d), lambda i: (i, 0))


def _p1_spec(d):
    return pl.BlockSpec((RBLK, d), lambda i: (i + GRID, 0))


def _full_spec(a, b):
    return pl.BlockSpec((a, b), lambda i: (0, 0))


def kernel(x, edge_index, W1, b1, g1, be1, W2, b2, g2, be2, W3, b3):
    src = edge_index[0]
    dst = edge_index[1]
    pad = EP - E
    src_p = jnp.concatenate([src, jnp.zeros((pad,), jnp.int32)])
    dst_p = jnp.concatenate([dst, jnp.full((pad,), N, jnp.int32)])
    idx_pk = jnp.stack([src_p.reshape(NW, NCHUNKS, CHUNK),
                        dst_p.reshape(NW, NCHUNKS, CHUNK)], axis=2)
    idx_pk = idx_pk.reshape(NW * NCHUNKS * 2, CHUNK)
    dst_rows = dst_p.reshape(NW * NCHUNKS, CHUNK)
    x_p = jnp.pad(x, ((0, NPAD - N), (0, 0)))
    b1r, g1r, be1r = b1.reshape(1, -1), g1.reshape(1, -1), be1.reshape(1, -1)
    b2r, g2r, be2r = b2.reshape(1, -1), g2.reshape(1, -1), be2.reshape(1, -1)
    b3r = b3.reshape(1, -1)

    _sc_hist, _sc_scatter128 = _sc_kernels()

    cnt = _sc_hist(dst_rows)

    h1p, dis8 = pl.pallas_call(
        _first_kernel,
        grid=(GRID,),
        in_specs=[_row_spec(128), _full_spec(128, 128),
                  _p0_spec(128), _p1_spec(128)],
        out_specs=[_row_spec(128), _row_spec(8)],
        out_shape=[jax.ShapeDtypeStruct((NPAD, 128), jnp.float32),
                   jax.ShapeDtypeStruct((NPAD, 8), jnp.float32)],
    )(x_p, W1, cnt, cnt)

    p1 = _sc_scatter128(idx_pk, h1p)

    h2p = pl.pallas_call(
        functools.partial(_mid_kernel, 128),
        grid=(GRID,),
        in_specs=[_p0_spec(128), _p1_spec(128), _row_spec(128),
                  _row_spec(8), _full_spec(128, 64),
                  _full_spec(1, 128), _full_spec(1, 128), _full_spec(1, 128)],
        out_specs=_row_spec(128),
        out_shape=jax.ShapeDtypeStruct((NPAD, 128), jnp.float32),
    )(p1, p1, h1p, dis8, W2, b1r, g1r, be1r)

    p2 = _sc_scatter128(idx_pk, h2p)

    h3p = pl.pallas_call(
        functools.partial(_mid_kernel, 64),
        grid=(GRID,),
        in_specs=[_p0_spec(128), _p1_spec(128), _row_spec(128),
                  _row_spec(8), _full_spec(64, 64),
                  _full_spec(1, 64), _full_spec(1, 64), _full_spec(1, 64)],
        out_specs=_row_spec(128),
        out_shape=jax.ShapeDtypeStruct((NPAD, 128), jnp.float32),
    )(p2, p2, h2p, dis8, W3, b2r, g2r, be2r)

    p3 = _sc_scatter128(idx_pk, h3p)

    out = pl.pallas_call(
        _last_kernel,
        grid=(GRID,),
        in_specs=[_p0_spec(128), _p1_spec(128), _row_spec(128),
                  _row_spec(8), _full_spec(1, 64)],
        out_specs=_row_spec(64),
        out_shape=jax.ShapeDtypeStruct((NPAD, 64), jnp.float32),
    )(p3, p3, h3p, dis8, b3r)

    return out[:N]


# speedup vs baseline: 1.3948x; 1.3948x over previous
import functools

import jax
import jax.numpy as jnp
from jax import lax
from jax.experimental import pallas as pl
from jax.experimental.pallas import tpu as pltpu
from jax.experimental.pallas import tpu_sc as plsc

N = 10000
E = 320000
NPAD = 10240
CHUNK = 128
NC = 2
NS = 16
NW = NC * NS
NCHUNKS = 79
NCH_H = 80
EPH = NW * NCH_H * CHUNK
EPW = NCHUNKS * CHUNK
EP = NW * EPW
RPT = NPAD // NS
EPS = 1e-5
RBLK = 2048
GRID = NPAD // RBLK

def _zero_fill(ref, d):
    z = jnp.zeros((16,), jnp.float32)
    for r in range(16):
        for c in range(d // 16):
            ref[r, pl.ds(c * 16, 16)] = z


def _zero_acc(acc, zbuf, sid, d):
    _zero_fill(zbuf, d)

    def zloop(i, _):
        pltpu.sync_copy(zbuf, acc.at[pl.ds(sid * RPT + i * 16, 16)])
        return 0

    lax.fori_loop(0, RPT // 16, zloop, 0)


@functools.cache
def _sc_kernels():
    mesh = plsc.VectorSubcoreMesh(core_axis_name="c", subcore_axis_name="s",
                                  num_cores=NC, num_subcores=NS)

    @functools.partial(
        pl.kernel,
        out_type=jax.ShapeDtypeStruct((NC * NPAD, 128), jnp.float32),
        mesh=mesh,
        scratch_types=[
            pltpu.VMEM_SHARED((NPAD, 128), jnp.float32),
            pltpu.VMEM((NCH_H, CHUNK), jnp.int32),
            pltpu.VMEM((CHUNK, 128), jnp.float32),
            pltpu.VMEM((16, 128), jnp.float32),
        ],
    )
    def _sc_hist(dst_hbm, out_hbm, acc, dbuf, ones, zbuf):
        cid = lax.axis_index("c")
        sid = lax.axis_index("s")
        _zero_acc(acc, zbuf, sid, 128)
        one = jnp.ones((16,), jnp.float32)
        for r in range(CHUNK):
            for c in range(8):
                ones[r, pl.ds(c * 16, 16)] = one
        wid = cid * NS + sid
        pltpu.sync_copy(dst_hbm.at[pl.ds(wid * NCH_H, NCH_H)], dbuf)
        plsc.subcore_barrier()

        def body(i, _):
            pltpu.sync_copy(ones, acc.at[dbuf.at[i]], add=True)
            return 0

        lax.fori_loop(0, NCH_H, body, 0)
        plsc.subcore_barrier()
        row0 = sid * RPT
        pltpu.sync_copy(acc.at[pl.ds(row0, RPT)],
                        out_hbm.at[pl.ds(cid * NPAD + row0, RPT)])

    def _make_sc_scatter(d):
        @functools.partial(
            pl.kernel,
            out_type=jax.ShapeDtypeStruct((NC * NPAD, d), jnp.float32),
            mesh=mesh,
            scratch_types=[
                pltpu.VMEM_SHARED((NPAD, d), jnp.float32),
                pltpu.VMEM((CHUNK,), jnp.int32),
                pltpu.VMEM((CHUNK,), jnp.int32),
                pltpu.VMEM((CHUNK, d), jnp.float32),
                pltpu.VMEM((16, d), jnp.float32),
                pltpu.SemaphoreType.DMA,
            ],
        )
        def _sc_scatter(src_hbm, dst_hbm, tbl_hbm, out_hbm,
                        acc, sidx, didx, rows, zbuf, gsem):
            cid = lax.axis_index("c")
            sid = lax.axis_index("s")
            _zero_acc(acc, zbuf, sid, d)
            plsc.subcore_barrier()
            base = (cid * NS + sid) * EPW

            def body(i, _):
                off = base + i * CHUNK
                pltpu.sync_copy(src_hbm.at[pl.ds(off, CHUNK)], sidx)
                pltpu.sync_copy(dst_hbm.at[pl.ds(off, CHUNK)], didx)
                pltpu.async_copy(tbl_hbm.at[sidx], rows, gsem).wait()
                pltpu.sync_copy(rows, acc.at[didx], add=True)
                return 0

            lax.fori_loop(0, NCHUNKS, body, 0)
            plsc.subcore_barrier()
            row0 = sid * RPT
            pltpu.sync_copy(acc.at[pl.ds(row0, RPT)],
                            out_hbm.at[pl.ds(cid * NPAD + row0, RPT)])

        return _sc_scatter

    return _sc_hist, _make_sc_scatter(128)


def _first_kernel(x_ref, w_ref, cnt0_ref, cnt1_ref, out_ref, dis_ref):
    c = cnt0_ref[...][:, 0:1] + cnt1_ref[...][:, 0:1]
    dis = lax.rsqrt(1.0 + c)
    dis_ref[...] = jnp.broadcast_to(dis, (RBLK, 8))
    out_ref[...] = dis * jnp.dot(x_ref[...], w_ref[...],
                                 preferred_element_type=jnp.float32,
                                 precision=lax.Precision.HIGHEST)


def _mid_kernel(dv, p0_ref, p1_ref, hp_ref, dis_ref,
                w_ref, b_ref, g_ref, be_ref, out_ref):
    dis = dis_ref[...][:, 0:1]
    v = (p0_ref[...] + p1_ref[...] + hp_ref[...])[:, :dv]
    conv = dis * v + b_ref[...]
    bnscale = 1.0 / jnp.sqrt(jnp.float32(1.0 + EPS))
    y = jnp.maximum(conv * bnscale * g_ref[...] + be_ref[...], 0.0)
    r = dis * jnp.dot(y, w_ref[...],
                      preferred_element_type=jnp.float32,
                      precision=lax.Precision.HIGHEST)
    out_ref[...] = jnp.concatenate(
        [r, jnp.zeros((RBLK, 128 - r.shape[1]), jnp.float32)], axis=1)


def _last_kernel(p0_ref, p1_ref, hp_ref, dis_ref, b_ref, out_ref):
    dis = dis_ref[...][:, 0:1]
    v = (p0_ref[...] + p1_ref[...] + hp_ref[...])[:, :64]
    out_ref[...] = dis * v + b_ref[...]


def _row_spec(d):
    return pl.BlockSpec((RBLK, d), lambda i: (i, 0))


def _p0_spec(d):
    return pl.BlockSpec((RBLK, d), lambda i: (i, 0))


def _p1_spec(d):
    return pl.BlockSpec((RBLK, d), lambda i: (i + GRID, 0))


def _full_spec(a, b):
    return pl.BlockSpec((a, b), lambda i: (0, 0))


def kernel(x, edge_index, W1, b1, g1, be1, W2, b2, g2, be2, W3, b3):
    src = edge_index[0]
    dst = edge_index[1]
    pad = EP - E
    src_p = jnp.concatenate([src, jnp.zeros((pad,), jnp.int32)])
    pad_dst = N + (jnp.arange(pad, dtype=jnp.int32) % (NPAD - N))
    dst_p = jnp.concatenate([dst, pad_dst])
    padh = EPH - E
    pad_dsth = N + (jnp.arange(padh, dtype=jnp.int32) % (NPAD - N))
    dst_rows = jnp.concatenate([dst, pad_dsth]).reshape(NW * NCH_H, CHUNK)
    x_p = jnp.pad(x, ((0, NPAD - N), (0, 0)))
    b1r, g1r, be1r = b1.reshape(1, -1), g1.reshape(1, -1), be1.reshape(1, -1)
    b2r, g2r, be2r = b2.reshape(1, -1), g2.reshape(1, -1), be2.reshape(1, -1)
    b3r = b3.reshape(1, -1)

    _sc_hist, _sc_scatter128 = _sc_kernels()

    cnt = _sc_hist(dst_rows)

    h1p, dis8 = pl.pallas_call(
        _first_kernel,
        grid=(GRID,),
        in_specs=[_row_spec(128), _full_spec(128, 128),
                  _p0_spec(128), _p1_spec(128)],
        out_specs=[_row_spec(128), _row_spec(8)],
        out_shape=[jax.ShapeDtypeStruct((NPAD, 128), jnp.float32),
                   jax.ShapeDtypeStruct((NPAD, 8), jnp.float32)],
    )(x_p, W1, cnt, cnt)

    p1 = _sc_scatter128(src_p, dst_p, h1p)

    h2p = pl.pallas_call(
        functools.partial(_mid_kernel, 128),
        grid=(GRID,),
        in_specs=[_p0_spec(128), _p1_spec(128), _row_spec(128),
                  _row_spec(8), _full_spec(128, 64),
                  _full_spec(1, 128), _full_spec(1, 128), _full_spec(1, 128)],
        out_specs=_row_spec(128),
        out_shape=jax.ShapeDtypeStruct((NPAD, 128), jnp.float32),
    )(p1, p1, h1p, dis8, W2, b1r, g1r, be1r)

    p2 = _sc_scatter128(src_p, dst_p, h2p)

    h3p = pl.pallas_call(
        functools.partial(_mid_kernel, 64),
        grid=(GRID,),
        in_specs=[_p0_spec(128), _p1_spec(128), _row_spec(128),
                  _row_spec(8), _full_spec(64, 64),
                  _full_spec(1, 64), _full_spec(1, 64), _full_spec(1, 64)],
        out_specs=_row_spec(128),
        out_shape=jax.ShapeDtypeStruct((NPAD, 128), jnp.float32),
    )(p2, p2, h2p, dis8, W3, b2r, g2r, be2r)

    p3 = _sc_scatter128(src_p, dst_p, h3p)

    out = pl.pallas_call(
        _last_kernel,
        grid=(GRID,),
        in_specs=[_p0_spec(128), _p1_spec(128), _row_spec(128),
                  _row_spec(8), _full_spec(1, 64)],
        out_specs=_row_spec(64),
        out_shape=jax.ShapeDtypeStruct((NPAD, 64), jnp.float32),
    )(p3, p3, h3p, dis8, b3r)

    return out[:N]


# speedup vs baseline: 2.0543x; 1.4729x over previous
import functools

import jax
import jax.numpy as jnp
from jax import lax
from jax.experimental import pallas as pl
from jax.experimental.pallas import tpu as pltpu
from jax.experimental.pallas import tpu_sc as plsc

N = 10000
E = 320000
NPAD = 10240
CHUNK = 128
NC = 2
NS = 16
NW = NC * NS
NCH0 = 116
NCH1 = 42
NCH_H = 80
EPH = NW * NCH_H * CHUNK
EP = NS * (NCH0 + NCH1) * CHUNK
RPT = NPAD // NS
EPS = 1e-5
RBLK = 2048
GRID = NPAD // RBLK

def _zero_fill(ref, d):
    z = jnp.zeros((16,), jnp.float32)
    for r in range(16):
        for c in range(d // 16):
            ref[r, pl.ds(c * 16, 16)] = z


def _zero_acc(acc, zbuf, sid, d):
    _zero_fill(zbuf, d)

    def zloop(i, _):
        pltpu.sync_copy(zbuf, acc.at[pl.ds(sid * RPT + i * 16, 16)])
        return 0

    lax.fori_loop(0, RPT // 16, zloop, 0)


@functools.cache
def _sc_kernels():
    mesh = plsc.VectorSubcoreMesh(core_axis_name="c", subcore_axis_name="s",
                                  num_cores=NC, num_subcores=NS)

    @functools.partial(
        pl.kernel,
        out_type=jax.ShapeDtypeStruct((NC * NPAD, 128), jnp.float32),
        mesh=mesh,
        scratch_types=[
            pltpu.VMEM_SHARED((NPAD, 128), jnp.float32),
            pltpu.VMEM((NCH_H, CHUNK), jnp.int32),
            pltpu.VMEM((CHUNK, 128), jnp.float32),
            pltpu.VMEM((16, 128), jnp.float32),
        ],
    )
    def _sc_hist(dst_hbm, out_hbm, acc, dbuf, ones, zbuf):
        cid = lax.axis_index("c")
        sid = lax.axis_index("s")
        _zero_acc(acc, zbuf, sid, 128)
        one = jnp.ones((16,), jnp.float32)
        for r in range(CHUNK):
            for c in range(8):
                ones[r, pl.ds(c * 16, 16)] = one
        wid = cid * NS + sid
        pltpu.sync_copy(dst_hbm.at[pl.ds(wid * NCH_H, NCH_H)], dbuf)
        plsc.subcore_barrier()

        def body(i, _):
            pltpu.sync_copy(ones, acc.at[dbuf.at[i]], add=True)
            return 0

        lax.fori_loop(0, NCH_H, body, 0)
        plsc.subcore_barrier()
        row0 = sid * RPT
        pltpu.sync_copy(acc.at[pl.ds(row0, RPT)],
                        out_hbm.at[pl.ds(cid * NPAD + row0, RPT)])

    def _make_sc_scatter(d):
        @functools.partial(
            pl.kernel,
            out_type=jax.ShapeDtypeStruct((NC * NPAD, d), jnp.float32),
            mesh=mesh,
            scratch_types=[
                pltpu.VMEM_SHARED((NPAD, d), jnp.float32),
                pltpu.VMEM((2, CHUNK), jnp.int32),
                pltpu.VMEM((2, CHUNK), jnp.int32),
                pltpu.VMEM((2, CHUNK, d), jnp.float32),
                pltpu.VMEM((16, d), jnp.float32),
                [pltpu.SemaphoreType.DMA] * 2,
            ],
        )
        def _sc_scatter(src_hbm, dst_hbm, tbl_hbm, out_hbm,
                        acc, sidx, didx, rows, zbuf, gsems):
            cid = lax.axis_index("c")
            sid = lax.axis_index("s")
            _zero_acc(acc, zbuf, sid, d)
            plsc.subcore_barrier()
            nch = jnp.where(cid == 0, NCH0, NCH1)
            base = jnp.where(cid == 0, sid * NCH0 * CHUNK,
                             NS * NCH0 * CHUNK + sid * NCH1 * CHUNK)

            def laf(b, i):
                off = base + i * CHUNK
                pltpu.sync_copy(src_hbm.at[pl.ds(off, CHUNK)], sidx.at[b])
                pltpu.sync_copy(dst_hbm.at[pl.ds(off, CHUNK)], didx.at[b])
                pltpu.async_copy(tbl_hbm.at[sidx.at[b]], rows.at[b], gsems[b])

            def drain(b):
                pltpu.make_async_copy(tbl_hbm.at[sidx.at[b]], rows.at[b],
                                      gsems[b]).wait()
                pltpu.sync_copy(rows.at[b], acc.at[didx.at[b]], add=True)

            for b in range(2):
                laf(b, b)

            def outer(g, _):
                for b in range(2):
                    drain(b)
                    laf(b, g * 2 + b + 2)
                return 0

            lax.fori_loop(0, nch // 2 - 1, outer, 0)
            for b in range(2):
                drain(b)
            plsc.subcore_barrier()
            row0 = sid * RPT
            pltpu.sync_copy(acc.at[pl.ds(row0, RPT)],
                            out_hbm.at[pl.ds(cid * NPAD + row0, RPT)])

        return _sc_scatter

    return _sc_hist, _make_sc_scatter(128)


def _first_kernel(x_ref, w_ref, cnt0_ref, cnt1_ref, out_ref, dis_ref):
    c = cnt0_ref[...][:, 0:1] + cnt1_ref[...][:, 0:1]
    dis = lax.rsqrt(1.0 + c)
    dis_ref[...] = jnp.broadcast_to(dis, (RBLK, 8))
    out_ref[...] = dis * jnp.dot(x_ref[...], w_ref[...],
                                 preferred_element_type=jnp.float32,
                                 precision=lax.Precision.HIGHEST)


def _mid_kernel(dv, p0_ref, p1_ref, hp_ref, dis_ref,
                w_ref, b_ref, g_ref, be_ref, out_ref):
    dis = dis_ref[...][:, 0:1]
    v = (p0_ref[...] + p1_ref[...] + hp_ref[...])[:, :dv]
    conv = dis * v + b_ref[...]
    bnscale = 1.0 / jnp.sqrt(jnp.float32(1.0 + EPS))
    y = jnp.maximum(conv * bnscale * g_ref[...] + be_ref[...], 0.0)
    r = dis * jnp.dot(y, w_ref[...],
                      preferred_element_type=jnp.float32,
                      precision=lax.Precision.HIGHEST)
    out_ref[...] = jnp.concatenate(
        [r, jnp.zeros((RBLK, 128 - r.shape[1]), jnp.float32)], axis=1)


def _last_kernel(p0_ref, p1_ref, hp_ref, dis_ref, b_ref, out_ref):
    dis = dis_ref[...][:, 0:1]
    v = (p0_ref[...] + p1_ref[...] + hp_ref[...])[:, :64]
    out_ref[...] = dis * v + b_ref[...]


def _row_spec(d):
    return pl.BlockSpec((RBLK, d), lambda i: (i, 0))


def _p0_spec(d):
    return pl.BlockSpec((RBLK, d), lambda i: (i, 0))


def _p1_spec(d):
    return pl.BlockSpec((RBLK, d), lambda i: (i + GRID, 0))


def _full_spec(a, b):
    return pl.BlockSpec((a, b), lambda i: (0, 0))


def kernel(x, edge_index, W1, b1, g1, be1, W2, b2, g2, be2, W3, b3):
    src = edge_index[0]
    dst = edge_index[1]
    pad = EP - E
    src_p = jnp.concatenate([src, jnp.zeros((pad,), jnp.int32)])
    pad_dst = N + (jnp.arange(pad, dtype=jnp.int32) % (NPAD - N))
    dst_p = jnp.concatenate([dst, pad_dst])
    padh = EPH - E
    pad_dsth = N + (jnp.arange(padh, dtype=jnp.int32) % (NPAD - N))
    dst_rows = jnp.concatenate([dst, pad_dsth]).reshape(NW * NCH_H, CHUNK)
    x_p = jnp.pad(x, ((0, NPAD - N), (0, 0)))
    b1r, g1r, be1r = b1.reshape(1, -1), g1.reshape(1, -1), be1.reshape(1, -1)
    b2r, g2r, be2r = b2.reshape(1, -1), g2.reshape(1, -1), be2.reshape(1, -1)
    b3r = b3.reshape(1, -1)

    _sc_hist, _sc_scatter128 = _sc_kernels()

    cnt = _sc_hist(dst_rows)

    h1p, dis8 = pl.pallas_call(
        _first_kernel,
        grid=(GRID,),
        in_specs=[_row_spec(128), _full_spec(128, 128),
                  _p0_spec(128), _p1_spec(128)],
        out_specs=[_row_spec(128), _row_spec(8)],
        out_shape=[jax.ShapeDtypeStruct((NPAD, 128), jnp.float32),
                   jax.ShapeDtypeStruct((NPAD, 8), jnp.float32)],
    )(x_p, W1, cnt, cnt)

    p1 = _sc_scatter128(src_p, dst_p, h1p)

    h2p = pl.pallas_call(
        functools.partial(_mid_kernel, 128),
        grid=(GRID,),
        in_specs=[_p0_spec(128), _p1_spec(128), _row_spec(128),
                  _row_spec(8), _full_spec(128, 64),
                  _full_spec(1, 128), _full_spec(1, 128), _full_spec(1, 128)],
        out_specs=_row_spec(128),
        out_shape=jax.ShapeDtypeStruct((NPAD, 128), jnp.float32),
    )(p1, p1, h1p, dis8, W2, b1r, g1r, be1r)

    p2 = _sc_scatter128(src_p, dst_p, h2p)

    h3p = pl.pallas_call(
        functools.partial(_mid_kernel, 64),
        grid=(GRID,),
        in_specs=[_p0_spec(128), _p1_spec(128), _row_spec(128),
                  _row_spec(8), _full_spec(64, 64),
                  _full_spec(1, 64), _full_spec(1, 64), _full_spec(1, 64)],
        out_specs=_row_spec(128),
        out_shape=jax.ShapeDtypeStruct((NPAD, 128), jnp.float32),
    )(p2, p2, h2p, dis8, W3, b2r, g2r, be2r)

    p3 = _sc_scatter128(src_p, dst_p, h3p)

    out = pl.pallas_call(
        _last_kernel,
        grid=(GRID,),
        in_specs=[_p0_spec(128), _p1_spec(128), _row_spec(128),
                  _row_spec(8), _full_spec(1, 64)],
        out_specs=_row_spec(64),
        out_shape=jax.ShapeDtypeStruct((NPAD, 64), jnp.float32),
    )(p3, p3, h3p, dis8, b3r)

    return out[:N]
